# Initial kernel scaffold; baseline (speedup 1.0000x reference)
import jax
import jax.numpy as jnp
from jax.experimental import pallas as pl


def kernel(sequence):
    raise NotImplementedError("write your pallas kernel here")


# speedup vs baseline: 16.6326x; 16.6326x over previous
import dataclasses
import functools

import jax
import jax.numpy as jnp
from jax import lax
from jax.experimental import pallas as pl
from jax.experimental.pallas import tpu as pltpu
from jax.experimental.pallas import tpu_sc as plsc

_A = 4
_K = 6
_B = 1024
_L = 4096
_NBINS = _A ** _K
_OUT_LEN = _L - _K + 1

_R1 = 16
_R2 = 64

_NW = 32
_ROWS_PER_WORKER = _B // _NW

_LANES = 16
_TAIL_START = (_OUT_LEN // _LANES) * _LANES
_TAIL_N = _OUT_LEN - _TAIL_START


def _shup(x, d, axis):
    return pltpu.roll(x, x.shape[axis] - d, axis)


def _argmax_tc_body(x_ref, o_ref):
    x = x_ref[...]
    r = x.shape[0]
    xs = _shup(x, 1, 2)
    b01 = xs > x
    m01 = jnp.where(b01, xs, x)
    i01 = b01.astype(jnp.int32)
    rm = _shup(m01, 2, 2)
    ri = _shup(i01, 2, 2)
    c = jnp.where(rm > m01, ri | 2, i01)
    lane = lax.broadcasted_iota(jnp.int32, (r, 128, 128), 2)
    g = jnp.take_along_axis(c, (lane % 32) * 4, axis=2)
    o_ref[...] = g[:, :, :32]


def _argmax_tc(seqv):
    return pl.pallas_call(
        _argmax_tc_body,
        grid=(_B // _R1,),
        in_specs=[pl.BlockSpec((_R1, 128, 128), lambda i: (i, 0, 0))],
        out_specs=pl.BlockSpec((_R1, 128, 32), lambda i: (i, 0, 0)),
        out_shape=jax.ShapeDtypeStruct((_B, 128, 32), jnp.int32),
    )(seqv)


def _encode_tc_body(c_ref, o_ref):
    c = c_ref[...]
    y1 = c * 4 + _shup(c, 1, 1)
    y2 = y1 * 16 + _shup(y1, 2, 1)
    km = y2 * 16 + _shup(y1, 4, 1)
    lane = lax.broadcasted_iota(jnp.int32, km.shape, 1)
    o_ref[...] = jnp.where(lane < _OUT_LEN, km, 0)


def _encode_tc(codes):
    return pl.pallas_call(
        _encode_tc_body,
        grid=(_B // _R2,),
        in_specs=[pl.BlockSpec((_R2, _L), lambda i: (i, 0))],
        out_specs=pl.BlockSpec((_R2, _L), lambda i: (i, 0)),
        out_shape=jax.ShapeDtypeStruct((_B, _L), jnp.int32),
    )(codes)


def _hist_sc(kmers):
    mesh = plsc.VectorSubcoreMesh(core_axis_name="c", subcore_axis_name="s")
    cp = pltpu.CompilerParams()
    if "needs_layout_passes" in pltpu.CompilerParams.__dataclass_fields__:
        cp = dataclasses.replace(cp, needs_layout_passes=False)

    @functools.partial(
        pl.kernel,
        compiler_params=cp,
        out_type=jax.ShapeDtypeStruct((_B, _NBINS), jnp.float32),
        mesh=mesh,
        scratch_types=[
            pltpu.VMEM((_L,), jnp.int32),
            pltpu.VMEM((_NBINS,), jnp.float32),
        ],
    )
    def k(kmers_hbm, out_hbm, kbuf, hist):
        wid = lax.axis_index("s") * 2 + lax.axis_index("c")
        zeros16 = jnp.zeros((_LANES,), jnp.float32)
        tail_valid = lax.iota(jnp.int32, _LANES) < _TAIL_N

        @pl.loop(0, _ROWS_PER_WORKER)
        def _row(rr):
            row = wid * _ROWS_PER_WORKER + rr
            pltpu.sync_copy(kmers_hbm.at[row], kbuf)

            @pl.loop(0, _NBINS, step=_LANES)
            def _zero(i):
                hist[pl.ds(i, _LANES)] = zeros16

            @pl.loop(0, _TAIL_START, step=_LANES)
            def _chunk(i):
                idx = kbuf[pl.ds(i, _LANES)]
                cnt, last = plsc.scan_count(idx)
                plsc.addupdate_scatter(
                    hist, [idx], cnt.astype(jnp.float32), mask=last)

            idx = kbuf[pl.ds(_TAIL_START, _LANES)]
            cnt, last = plsc.scan_count(idx, mask=tail_valid)
            plsc.addupdate_scatter(
                hist, [idx], cnt.astype(jnp.float32), mask=last)

            pltpu.sync_copy(hist, out_hbm.at[row])

    return k(kmers)


def kernel(sequence):
    seqv = sequence.reshape(_B, 128, 128)
    codes = _argmax_tc(seqv).reshape(_B, _L)
    kmers = _encode_tc(codes)
    return _hist_sc(kmers)


# speedup vs baseline: 25.2613x; 1.5188x over previous
import dataclasses
import functools

import jax
import jax.numpy as jnp
import numpy as np
from jax import lax
from jax.experimental import pallas as pl
from jax.experimental.pallas import tpu as pltpu
from jax.experimental.pallas import tpu_sc as plsc

_A = 4
_K = 6
_B = 1024
_L = 4096
_NBINS = _A ** _K
_OUT_LEN = _L - _K + 1

_R1 = 16

_NW = 32
_ROWS_PER_WORKER = _B // _NW

_LANES = 16
_TAIL_START = (_OUT_LEN // _LANES) * _LANES
_TAIL_N = _OUT_LEN - _TAIL_START


def _shup(x, d, axis):
    return pltpu.roll(x, x.shape[axis] - d, axis)


def _window_weights():
    w1 = np.zeros((128, 32), np.float32)
    w2 = np.zeros((128, 32), np.float32)
    for j in range(32):
        for d in range(_K):
            w = float(_A ** (_K - 1 - d))
            if j + d < 32:
                w1[4 * (j + d), j] = w
            else:
                w2[4 * (j + d - 32), j] = w
    return jnp.asarray(w1, jnp.bfloat16), jnp.asarray(w2, jnp.bfloat16)


def _kmer_tc_body(x_ref, w1_ref, w2_ref, o_ref):
    x = x_ref[...]
    r = x.shape[0]
    xs = _shup(x, 1, 2)
    m01 = jnp.maximum(x, xs)
    b01 = m01 > x
    rm = _shup(m01, 2, 2)
    h = rm > m01
    i01 = jnp.where(b01, 1.0, 0.0)
    ri2 = _shup(i01, 2, 2) + 2.0
    cf = jnp.where(h, ri2, i01)
    cbf = cf.astype(jnp.bfloat16)
    cup = pltpu.roll(cbf, 127, 1)
    a = lax.dot_general(
        cbf.reshape(r * 128, 128), w1_ref[...],
        (((1,), (0,)), ((), ())), preferred_element_type=jnp.float32)
    b = lax.dot_general(
        cup.reshape(r * 128, 128), w2_ref[...],
        (((1,), (0,)), ((), ())), preferred_element_type=jnp.float32)
    o_ref[...] = (a + b).reshape(r, 128, 32).astype(jnp.int32)


def _kmer_tc(seqv):
    w1, w2 = _window_weights()
    return pl.pallas_call(
        _kmer_tc_body,
        grid=(_B // _R1,),
        in_specs=[
            pl.BlockSpec((_R1, 128, 128), lambda i: (i, 0, 0)),
            pl.BlockSpec((128, 32), lambda i: (0, 0)),
            pl.BlockSpec((128, 32), lambda i: (0, 0)),
        ],
        out_specs=pl.BlockSpec((_R1, 128, 32), lambda i: (i, 0, 0)),
        out_shape=jax.ShapeDtypeStruct((_B, 128, 32), jnp.int32),
    )(seqv, w1, w2)


_RB = 4
_NBATCH = _ROWS_PER_WORKER // _RB


def _hist_sc(kmers):
    mesh = plsc.VectorSubcoreMesh(core_axis_name="c", subcore_axis_name="s")
    cp = pltpu.CompilerParams()
    if "needs_layout_passes" in pltpu.CompilerParams.__dataclass_fields__:
        cp = dataclasses.replace(cp, needs_layout_passes=False)

    @functools.partial(
        pl.kernel,
        compiler_params=cp,
        out_type=jax.ShapeDtypeStruct((_B, _NBINS), jnp.float32),
        mesh=mesh,
        scratch_types=[
            pltpu.VMEM((_RB, _L), jnp.int32),
            pltpu.VMEM((_RB, _L), jnp.int32),
            pltpu.VMEM((_RB, _NBINS), jnp.float32),
            pltpu.VMEM((_RB, _NBINS), jnp.float32),
            pltpu.SemaphoreType.DMA,
            pltpu.SemaphoreType.DMA,
            pltpu.SemaphoreType.DMA,
            pltpu.SemaphoreType.DMA,
        ],
    )
    def k(kmers_hbm, out_hbm, kb0, kb1, h0, h1, si0, si1, so0, so1):
        wid = lax.axis_index("s") * 2 + lax.axis_index("c")
        base = wid * _ROWS_PER_WORKER
        kbs = (kb0, kb1)
        hs = (h0, h1)
        sis = (si0, si1)
        sos = (so0, so1)
        zeros16 = jnp.zeros((_LANES,), jnp.float32)
        tail_valid = lax.iota(jnp.int32, _LANES) < _TAIL_N

        pltpu.async_copy(kmers_hbm.at[pl.ds(base, _RB)], kb0, si0)
        pltpu.async_copy(kmers_hbm.at[pl.ds(base + _RB, _RB)], kb1, si1)

        @pl.loop(0, _NBATCH, step=2)
        def _batch(bb):
            for p in (0, 1):
                b = bb + p
                row0 = base + b * _RB
                pltpu.make_async_copy(
                    kmers_hbm.at[pl.ds(row0, _RB)], kbs[p], sis[p]).wait()

                @pl.when(b >= 2)
                def _wait_out():
                    pltpu.make_async_copy(
                        hs[p], out_hbm.at[pl.ds(row0 - 2 * _RB, _RB)],
                        sos[p]).wait()

                for r2 in range(_RB):
                    rsplat = jnp.full((_LANES,), r2, jnp.int32)

                    @plsc.parallel_loop(0, _NBINS, _LANES, unroll=8)
                    def _zero(i):
                        hs[p][r2, pl.ds(i, _LANES)] = zeros16

                    @plsc.parallel_loop(0, _TAIL_START, _LANES, unroll=8)
                    def _chunk(i):
                        idx = kbs[p][r2, pl.ds(i, _LANES)]
                        cnt, last = plsc.scan_count(idx)
                        plsc.addupdate_scatter(
                            hs[p], [rsplat, idx],
                            cnt.astype(jnp.float32), mask=last)

                    idx = kbs[p][r2, pl.ds(_TAIL_START, _LANES)]
                    cnt, last = plsc.scan_count(idx, mask=tail_valid)
                    plsc.addupdate_scatter(
                        hs[p], [rsplat, idx],
                        cnt.astype(jnp.float32), mask=last)

                pltpu.async_copy(hs[p], out_hbm.at[pl.ds(row0, _RB)], sos[p])

                @pl.when(b + 2 < _NBATCH)
                def _next_in():
                    pltpu.async_copy(
                        kmers_hbm.at[pl.ds(row0 + 2 * _RB, _RB)],
                        kbs[p], sis[p])

        pltpu.make_async_copy(
            h0, out_hbm.at[pl.ds(base + (_NBATCH - 2) * _RB, _RB)], so0).wait()
        pltpu.make_async_copy(
            h1, out_hbm.at[pl.ds(base + (_NBATCH - 1) * _RB, _RB)], so1).wait()

    return k(kmers)


def kernel(sequence):
    seqv = sequence.reshape(_B, 128, 128)
    kmers = _kmer_tc(seqv).reshape(_B, _L)
    return _hist_sc(kmers)


# speedup vs baseline: 91.8941x; 3.6377x over previous
import dataclasses
import functools

import jax
import jax.numpy as jnp
from jax import lax
from jax.experimental import pallas as pl
from jax.experimental.pallas import tpu as pltpu
from jax.experimental.pallas import tpu_sc as plsc

_A = 4
_K = 6
_B = 1024
_L = 4096
_NBINS = _A ** _K
_OUT_LEN = _L - _K + 1

_R1 = 16

_NW = 32
_ROWS_PER_WORKER = _B // _NW

_LANES = 16
_TAIL_START = (_OUT_LEN // _LANES) * _LANES
_TAIL_N = _OUT_LEN - _TAIL_START

_RB = 4
_NBATCH = _ROWS_PER_WORKER // _RB


def _shup(x, d, axis):
    return pltpu.roll(x, x.shape[axis] - d, axis)


def _kmer_tc_body(x_ref, o_ref):
    x = x_ref[...]
    x0 = x[:, 0, :]
    x1 = x[:, 1, :]
    x2 = x[:, 2, :]
    x3 = x[:, 3, :]
    m01 = jnp.maximum(x0, x1)
    b01 = m01 > x0
    m23 = jnp.maximum(x2, x3)
    h = m23 > m01
    i01 = jnp.where(b01, 1, 0)
    i23 = jnp.where(m23 > x2, 3, 2)
    c = jnp.where(h, i23, i01)
    y1 = c * 4 + _shup(c, 1, 1)
    y2 = y1 * 16 + _shup(y1, 2, 1)
    km = y2 * 16 + _shup(y1, 4, 1)
    o_ref[...] = km


def _kmer_tc(xt):
    return pl.pallas_call(
        _kmer_tc_body,
        grid=(_B // _R1,),
        in_specs=[pl.BlockSpec((_R1, _A, _L), lambda i: (i, 0, 0))],
        out_specs=pl.BlockSpec((_R1, _L), lambda i: (i, 0)),
        out_shape=jax.ShapeDtypeStruct((_B, _L), jnp.int32),
    )(xt)


def _hist_sc(kmers):
    mesh = plsc.VectorSubcoreMesh(core_axis_name="c", subcore_axis_name="s")
    cp = pltpu.CompilerParams()
    if "needs_layout_passes" in pltpu.CompilerParams.__dataclass_fields__:
        cp = dataclasses.replace(cp, needs_layout_passes=False)

    @functools.partial(
        pl.kernel,
        compiler_params=cp,
        out_type=jax.ShapeDtypeStruct((_B, _NBINS), jnp.float32),
        mesh=mesh,
        scratch_types=[
            pltpu.VMEM((_RB, _L), jnp.int32),
            pltpu.VMEM((_RB, _L), jnp.int32),
            pltpu.VMEM((_RB, _NBINS), jnp.float32),
            pltpu.VMEM((_RB, _NBINS), jnp.float32),
            pltpu.SemaphoreType.DMA,
            pltpu.SemaphoreType.DMA,
            pltpu.SemaphoreType.DMA,
            pltpu.SemaphoreType.DMA,
        ],
    )
    def k(kmers_hbm, out_hbm, kb0, kb1, h0, h1, si0, si1, so0, so1):
        wid = lax.axis_index("s") * 2 + lax.axis_index("c")
        base = wid * _ROWS_PER_WORKER
        kbs = (kb0, kb1)
        hs = (h0, h1)
        sis = (si0, si1)
        sos = (so0, so1)
        zeros16 = jnp.zeros((_LANES,), jnp.float32)
        tail_valid = lax.iota(jnp.int32, _LANES) < _TAIL_N

        pltpu.async_copy(kmers_hbm.at[pl.ds(base, _RB)], kb0, si0)
        pltpu.async_copy(kmers_hbm.at[pl.ds(base + _RB, _RB)], kb1, si1)

        @pl.loop(0, _NBATCH, step=2)
        def _batch(bb):
            for p in (0, 1):
                b = bb + p
                row0 = base + b * _RB
                pltpu.make_async_copy(
                    kmers_hbm.at[pl.ds(row0, _RB)], kbs[p], sis[p]).wait()

                @pl.when(b >= 2)
                def _wait_out():
                    pltpu.make_async_copy(
                        hs[p], out_hbm.at[pl.ds(row0 - 2 * _RB, _RB)],
                        sos[p]).wait()

                for r2 in range(_RB):
                    rsplat = jnp.full((_LANES,), r2, jnp.int32)

                    @plsc.parallel_loop(0, _NBINS, _LANES, unroll=8)
                    def _zero(i):
                        hs[p][r2, pl.ds(i, _LANES)] = zeros16

                    @plsc.parallel_loop(0, _TAIL_START, _LANES, unroll=8)
                    def _chunk(i):
                        idx = kbs[p][r2, pl.ds(i, _LANES)]
                        cnt, last = plsc.scan_count(idx)
                        plsc.addupdate_scatter(
                            hs[p], [rsplat, idx],
                            cnt.astype(jnp.float32), mask=last)

                    idx = kbs[p][r2, pl.ds(_TAIL_START, _LANES)]
                    cnt, last = plsc.scan_count(idx, mask=tail_valid)
                    plsc.addupdate_scatter(
                        hs[p], [rsplat, idx],
                        cnt.astype(jnp.float32), mask=last)

                pltpu.async_copy(hs[p], out_hbm.at[pl.ds(row0, _RB)], sos[p])

                @pl.when(b + 2 < _NBATCH)
                def _next_in():
                    pltpu.async_copy(
                        kmers_hbm.at[pl.ds(row0 + 2 * _RB, _RB)],
                        kbs[p], sis[p])

        pltpu.make_async_copy(
            h0, out_hbm.at[pl.ds(base + (_NBATCH - 2) * _RB, _RB)], so0).wait()
        pltpu.make_async_copy(
            h1, out_hbm.at[pl.ds(base + (_NBATCH - 1) * _RB, _RB)], so1).wait()

    return k(kmers)


def kernel(sequence):
    xt = jnp.transpose(sequence, (0, 2, 1))
    kmers = _kmer_tc(xt)
    return _hist_sc(kmers)
